# baseline (speedup 1.0000x reference)
import functools

import jax
import jax.numpy as jnp
from jax import lax
from jax.experimental import pallas as pl
from jax.experimental.pallas import tpu as pltpu
from jax.experimental.pallas import tpu_sc as plsc

B = 16384
EMB = 16
H1, H2, H3 = 16, 8, 4

NC, NS, L = 2, 16, 16
NW = NC * NS
BPW = B // NW
G = BPW // L
RPW = BPW * EMB

OFF_W1 = 0
OFF_B1 = OFF_W1 + 2 * EMB * H1
OFF_W2 = OFF_B1 + H1
OFF_B2 = OFF_W2 + H1 * H2
OFF_W3 = OFF_B2 + H2
OFF_B3 = OFF_W3 + H2 * H3
OFF_WP = OFF_B3 + H3
OFF_BP = OFF_WP + EMB + H3
NPAR = 728


def _tile_body(idxu_h, idxi_h, uM_h, iM_h, uG_h, iG_h, par_h, out_h,
               idxu_v, idxi_v, uM_v, iM_v, uG_v, iG_v, par_v, par_s, out_v,
               sem):
    wid = lax.axis_index("s") * NC + lax.axis_index("c")
    base = pl.multiple_of(wid * RPW, RPW)

    pltpu.sync_copy(idxu_h.at[pl.ds(base, RPW)], idxu_v)
    pltpu.sync_copy(idxi_h.at[pl.ds(base, RPW)], idxi_v)
    c1 = pltpu.async_copy(uM_h.at[idxu_v], uM_v, sem)
    c2 = pltpu.async_copy(iM_h.at[idxi_v], iM_v, sem)
    c3 = pltpu.async_copy(uG_h.at[idxu_v], uG_v, sem)
    c4 = pltpu.async_copy(iG_h.at[idxi_v], iG_v, sem)
    pltpu.sync_copy(par_h, par_v)
    pltpu.sync_copy(par_v, par_s)
    c1.wait()
    c2.wait()
    c3.wait()
    c4.wait()

    iota = lax.iota(jnp.int32, L)

    def group(g, carry):
        start = pl.multiple_of(g * L, L)
        flat = (start + iota) * EMB

        xs = [plsc.load_gather(uM_v, [flat + k]) for k in range(EMB)]
        xs += [plsc.load_gather(iM_v, [flat + k]) for k in range(EMB)]

        h1 = []
        for j in range(H1):
            z = xs[0] * par_s[OFF_W1 + j]
            for k in range(1, 2 * EMB):
                z = z + xs[k] * par_s[OFF_W1 + k * H1 + j]
            h1.append(jnp.maximum(z + par_s[OFF_B1 + j], 0.0))

        h2 = []
        for j in range(H2):
            z = h1[0] * par_s[OFF_W2 + j]
            for k in range(1, H1):
                z = z + h1[k] * par_s[OFF_W2 + k * H2 + j]
            h2.append(jnp.maximum(z + par_s[OFF_B2 + j], 0.0))

        h3 = []
        for j in range(H3):
            z = h2[0] * par_s[OFF_W3 + j]
            for k in range(1, H2):
                z = z + h2[k] * par_s[OFF_W3 + k * H3 + j]
            h3.append(jnp.maximum(z + par_s[OFF_B3 + j], 0.0))

        z = None
        for k in range(EMB):
            gm = (plsc.load_gather(uG_v, [flat + k])
                  * plsc.load_gather(iG_v, [flat + k]))
            t = gm * par_s[OFF_WP + k]
            z = t if z is None else z + t
        for k in range(H3):
            z = z + h3[k] * par_s[OFF_WP + EMB + k]
        z = z + par_s[OFF_BP]
        out_v[pl.ds(start, L)] = 1.0 / (1.0 + jnp.exp(-z))
        return carry

    lax.fori_loop(0, G, group, 0)
    pltpu.sync_copy(out_v, out_h.at[pl.ds(pl.multiple_of(wid * BPW, BPW), BPW)])


@functools.partial(
    pl.kernel,
    mesh=plsc.VectorSubcoreMesh(core_axis_name="c", subcore_axis_name="s"),
    compiler_params=pltpu.CompilerParams(
        needs_layout_passes=False, use_tc_tiling_on_sc=True),
    out_type=jax.ShapeDtypeStruct((B,), jnp.float32),
    scratch_types=[
        pltpu.VMEM((RPW,), jnp.int32),
        pltpu.VMEM((RPW,), jnp.int32),
        pltpu.VMEM((RPW,), jnp.float32),
        pltpu.VMEM((RPW,), jnp.float32),
        pltpu.VMEM((RPW,), jnp.float32),
        pltpu.VMEM((RPW,), jnp.float32),
        pltpu.VMEM_SHARED((NPAR,), jnp.float32),
        pltpu.SMEM((NPAR,), jnp.float32),
        pltpu.VMEM((BPW,), jnp.float32),
        pltpu.SemaphoreType.DMA,
    ],
)
def _neumf_sc(idxu_h, idxi_h, uM_h, iM_h, uG_h, iG_h, par_h, out_h,
              idxu_v, idxi_v, uM_v, iM_v, uG_v, iG_v, par_v, par_s, out_v,
              sem):
    _tile_body(idxu_h, idxi_h, uM_h, iM_h, uG_h, iG_h, par_h, out_h,
               idxu_v, idxi_v, uM_v, iM_v, uG_v, iG_v, par_v, par_s, out_v,
               sem)


def kernel(userID, itemID, user_emb_MLP, item_emb_MLP, user_emb_GMF,
           item_emb_GMF, W1, b1, W2, b2, W3, b3, Wp, bp):
    uid = userID.reshape(-1).astype(jnp.int32)
    iid = itemID.reshape(-1).astype(jnp.int32)
    f16 = jnp.arange(EMB, dtype=jnp.int32)
    idxu = (uid[:, None] * EMB + f16[None, :]).reshape(-1)
    idxi = (iid[:, None] * EMB + f16[None, :]).reshape(-1)
    params = jnp.concatenate([
        W1.reshape(-1), b1, W2.reshape(-1), b2, W3.reshape(-1), b3,
        Wp.reshape(-1), bp,
        jnp.zeros((NPAR - OFF_BP - 1,), jnp.float32),
    ]).astype(jnp.float32)
    one = 1.0 + b1[0] * 0.0
    out = _neumf_sc(idxu, idxi,
                    user_emb_MLP.reshape(-1) * one,
                    item_emb_MLP.reshape(-1) * one,
                    user_emb_GMF.reshape(-1) * one,
                    item_emb_GMF.reshape(-1) * one,
                    params)
    return out.reshape(B, 1)


# speedup vs baseline: 1.6445x; 1.6445x over previous
import functools

import jax
import jax.numpy as jnp
from jax import lax
from jax.experimental import pallas as pl
from jax.experimental.pallas import tpu as pltpu
from jax.experimental.pallas import tpu_sc as plsc

B = 16384
EMB = 16
H1, H2, H3 = 16, 8, 4

NC, NS, L = 2, 16, 16
NW = NC * NS
BPW = B // NW
G = BPW // L

OFF_W1 = 0
OFF_B1 = OFF_W1 + 2 * EMB * H1
OFF_W2 = OFF_B1 + H1
OFF_B2 = OFF_W2 + H1 * H2
OFF_W3 = OFF_B2 + H2
OFF_B3 = OFF_W3 + H2 * H3
OFF_WP = OFF_B3 + H3
OFF_BP = OFF_WP + EMB + H3
NPAR = 728


def _tile_body(uid_h, iid_h, uM_h, iM_h, uG_h, iG_h, par_h, out_h,
               uid_v, iid_v, uM_v, iM_v, uG_v, iG_v, par_v, par_s, out_v,
               sem):
    wid = lax.axis_index("s") * NC + lax.axis_index("c")
    base = pl.multiple_of(wid * BPW, BPW)

    pltpu.sync_copy(uid_h.at[pl.ds(base, BPW)], uid_v)
    pltpu.sync_copy(iid_h.at[pl.ds(base, BPW)], iid_v)
    c1 = pltpu.async_copy(uM_h.at[uid_v], uM_v, sem)
    c2 = pltpu.async_copy(iM_h.at[iid_v], iM_v, sem)
    c3 = pltpu.async_copy(uG_h.at[uid_v], uG_v, sem)
    c4 = pltpu.async_copy(iG_h.at[iid_v], iG_v, sem)
    pltpu.sync_copy(par_h, par_v)
    pltpu.sync_copy(par_v, par_s)
    c1.wait()
    c2.wait()
    c3.wait()
    c4.wait()

    iota = lax.iota(jnp.int32, L)

    def group(g, carry):
        start = pl.multiple_of(g * L, L)
        row = start + iota

        cols = [jnp.full((L,), k, jnp.int32) for k in range(EMB)]
        xs = [plsc.load_gather(uM_v, [row, cols[k]]) for k in range(EMB)]
        xs += [plsc.load_gather(iM_v, [row, cols[k]]) for k in range(EMB)]

        h1 = []
        for j in range(H1):
            z = xs[0] * par_s[OFF_W1 + j]
            for k in range(1, 2 * EMB):
                z = z + xs[k] * par_s[OFF_W1 + k * H1 + j]
            h1.append(jnp.maximum(z + par_s[OFF_B1 + j], 0.0))

        h2 = []
        for j in range(H2):
            z = h1[0] * par_s[OFF_W2 + j]
            for k in range(1, H1):
                z = z + h1[k] * par_s[OFF_W2 + k * H2 + j]
            h2.append(jnp.maximum(z + par_s[OFF_B2 + j], 0.0))

        h3 = []
        for j in range(H3):
            z = h2[0] * par_s[OFF_W3 + j]
            for k in range(1, H2):
                z = z + h2[k] * par_s[OFF_W3 + k * H3 + j]
            h3.append(jnp.maximum(z + par_s[OFF_B3 + j], 0.0))

        z = None
        for k in range(EMB):
            gm = (plsc.load_gather(uG_v, [row, cols[k]])
                  * plsc.load_gather(iG_v, [row, cols[k]]))
            t = gm * par_s[OFF_WP + k]
            z = t if z is None else z + t
        for k in range(H3):
            z = z + h3[k] * par_s[OFF_WP + EMB + k]
        z = z + par_s[OFF_BP]
        out_v[pl.ds(start, L)] = 1.0 / (1.0 + jnp.exp(-z))
        return carry

    lax.fori_loop(0, G, group, 0)
    pltpu.sync_copy(out_v, out_h.at[pl.ds(base, BPW)])


@functools.partial(
    pl.kernel,
    mesh=plsc.VectorSubcoreMesh(core_axis_name="c", subcore_axis_name="s"),
    compiler_params=pltpu.CompilerParams(
        needs_layout_passes=False, use_tc_tiling_on_sc=False),
    out_type=jax.ShapeDtypeStruct((B,), jnp.float32),
    scratch_types=[
        pltpu.VMEM((BPW,), jnp.int32),
        pltpu.VMEM((BPW,), jnp.int32),
        pltpu.VMEM((BPW, EMB), jnp.float32),
        pltpu.VMEM((BPW, EMB), jnp.float32),
        pltpu.VMEM((BPW, EMB), jnp.float32),
        pltpu.VMEM((BPW, EMB), jnp.float32),
        pltpu.VMEM_SHARED((NPAR,), jnp.float32),
        pltpu.SMEM((NPAR,), jnp.float32),
        pltpu.VMEM((BPW,), jnp.float32),
        pltpu.SemaphoreType.DMA,
    ],
)
def _neumf_sc(uid_h, iid_h, uM_h, iM_h, uG_h, iG_h, par_h, out_h,
              uid_v, iid_v, uM_v, iM_v, uG_v, iG_v, par_v, par_s, out_v,
              sem):
    _tile_body(uid_h, iid_h, uM_h, iM_h, uG_h, iG_h, par_h, out_h,
               uid_v, iid_v, uM_v, iM_v, uG_v, iG_v, par_v, par_s, out_v,
               sem)


def kernel(userID, itemID, user_emb_MLP, item_emb_MLP, user_emb_GMF,
           item_emb_GMF, W1, b1, W2, b2, W3, b3, Wp, bp):
    uid = userID.reshape(-1).astype(jnp.int32)
    iid = itemID.reshape(-1).astype(jnp.int32)
    params = jnp.concatenate([
        W1.reshape(-1), b1, W2.reshape(-1), b2, W3.reshape(-1), b3,
        Wp.reshape(-1), bp,
        jnp.zeros((NPAR - OFF_BP - 1,), jnp.float32),
    ]).astype(jnp.float32)
    out = _neumf_sc(uid, iid, user_emb_MLP, item_emb_MLP, user_emb_GMF,
                    item_emb_GMF, params)
    return out.reshape(B, 1)


# speedup vs baseline: 1.6779x; 1.0203x over previous
import functools

import jax
import jax.numpy as jnp
from jax import lax
from jax.experimental import pallas as pl
from jax.experimental.pallas import tpu as pltpu
from jax.experimental.pallas import tpu_sc as plsc

B = 16384
EMB = 16
H1, H2, H3 = 16, 8, 4

NC, NS, L = 2, 16, 16
NW = NC * NS
BPW = B // NW
G = BPW // L

OFF_W1 = 0
OFF_B1 = OFF_W1 + 2 * EMB * H1
OFF_W2 = OFF_B1 + H1
OFF_B2 = OFF_W2 + H1 * H2
OFF_W3 = OFF_B2 + H2
OFF_B3 = OFF_W3 + H2 * H3
OFF_WP = OFF_B3 + H3
OFF_BP = OFF_WP + EMB + H3
NPAR = 728


def _tile_body(uid_h, iid_h, uM_h, iM_h, uG_h, iG_h, par_h, out_h,
               uid_v, iid_v, uM_v, iM_v, uG_v, iG_v, par_v, par_s, out_v,
               sem):
    wid = lax.axis_index("s") * NC + lax.axis_index("c")
    base = pl.multiple_of(wid * BPW, BPW)

    pltpu.sync_copy(uid_h.at[pl.ds(base, BPW)], uid_v)
    pltpu.sync_copy(iid_h.at[pl.ds(base, BPW)], iid_v)
    c1 = pltpu.async_copy(uM_h.at[uid_v], uM_v, sem)
    c2 = pltpu.async_copy(iM_h.at[iid_v], iM_v, sem)
    c3 = pltpu.async_copy(uG_h.at[uid_v], uG_v, sem)
    c4 = pltpu.async_copy(iG_h.at[iid_v], iG_v, sem)
    pltpu.sync_copy(par_h, par_v)
    pltpu.sync_copy(par_v, par_s)
    c1.wait()
    c2.wait()
    c3.wait()
    c4.wait()

    iota = lax.iota(jnp.int32, L)

    def group(g, carry):
        start = pl.multiple_of(g * L, L)
        row = start + iota

        cols = [jnp.full((L,), k, jnp.int32) for k in range(EMB)]
        acc1 = [None] * H1
        for k in range(2 * EMB):
            if k < EMB:
                x = plsc.load_gather(uM_v, [row, cols[k]])
            else:
                x = plsc.load_gather(iM_v, [row, cols[k - EMB]])
            for j in range(H1):
                t = x * par_s[OFF_W1 + k * H1 + j]
                acc1[j] = t if acc1[j] is None else acc1[j] + t
        h1 = [jnp.maximum(acc1[j] + par_s[OFF_B1 + j], 0.0)
              for j in range(H1)]

        acc2 = [None] * H2
        for k in range(H1):
            for j in range(H2):
                t = h1[k] * par_s[OFF_W2 + k * H2 + j]
                acc2[j] = t if acc2[j] is None else acc2[j] + t
        h2 = [jnp.maximum(acc2[j] + par_s[OFF_B2 + j], 0.0)
              for j in range(H2)]

        acc3 = [None] * H3
        for k in range(H2):
            for j in range(H3):
                t = h2[k] * par_s[OFF_W3 + k * H3 + j]
                acc3[j] = t if acc3[j] is None else acc3[j] + t
        h3 = [jnp.maximum(acc3[j] + par_s[OFF_B3 + j], 0.0)
              for j in range(H3)]

        z = None
        for k in range(EMB):
            gm = (plsc.load_gather(uG_v, [row, cols[k]])
                  * plsc.load_gather(iG_v, [row, cols[k]]))
            t = gm * par_s[OFF_WP + k]
            z = t if z is None else z + t
        for k in range(H3):
            z = z + h3[k] * par_s[OFF_WP + EMB + k]
        z = z + par_s[OFF_BP]
        out_v[pl.ds(start, L)] = 1.0 / (1.0 + jnp.exp(-z))
        return carry

    lax.fori_loop(0, G, group, 0)
    pltpu.sync_copy(out_v, out_h.at[pl.ds(base, BPW)])


@functools.partial(
    pl.kernel,
    mesh=plsc.VectorSubcoreMesh(core_axis_name="c", subcore_axis_name="s"),
    compiler_params=pltpu.CompilerParams(
        needs_layout_passes=False, use_tc_tiling_on_sc=False),
    out_type=jax.ShapeDtypeStruct((B,), jnp.float32),
    scratch_types=[
        pltpu.VMEM((BPW,), jnp.int32),
        pltpu.VMEM((BPW,), jnp.int32),
        pltpu.VMEM((BPW, EMB), jnp.float32),
        pltpu.VMEM((BPW, EMB), jnp.float32),
        pltpu.VMEM((BPW, EMB), jnp.float32),
        pltpu.VMEM((BPW, EMB), jnp.float32),
        pltpu.VMEM_SHARED((NPAR,), jnp.float32),
        pltpu.SMEM((NPAR,), jnp.float32),
        pltpu.VMEM((BPW,), jnp.float32),
        pltpu.SemaphoreType.DMA,
    ],
)
def _neumf_sc(uid_h, iid_h, uM_h, iM_h, uG_h, iG_h, par_h, out_h,
              uid_v, iid_v, uM_v, iM_v, uG_v, iG_v, par_v, par_s, out_v,
              sem):
    _tile_body(uid_h, iid_h, uM_h, iM_h, uG_h, iG_h, par_h, out_h,
               uid_v, iid_v, uM_v, iM_v, uG_v, iG_v, par_v, par_s, out_v,
               sem)


def kernel(userID, itemID, user_emb_MLP, item_emb_MLP, user_emb_GMF,
           item_emb_GMF, W1, b1, W2, b2, W3, b3, Wp, bp):
    uid = userID.reshape(-1).astype(jnp.int32)
    iid = itemID.reshape(-1).astype(jnp.int32)
    params = jnp.concatenate([
        W1.reshape(-1), b1, W2.reshape(-1), b2, W3.reshape(-1), b3,
        Wp.reshape(-1), bp,
        jnp.zeros((NPAR - OFF_BP - 1,), jnp.float32),
    ]).astype(jnp.float32)
    out = _neumf_sc(uid, iid, user_emb_MLP, item_emb_MLP, user_emb_GMF,
                    item_emb_GMF, params)
    return out.reshape(B, 1)


# speedup vs baseline: 1.6810x; 1.0018x over previous
import functools

import jax
import jax.numpy as jnp
from jax import lax
from jax.experimental import pallas as pl
from jax.experimental.pallas import tpu as pltpu
from jax.experimental.pallas import tpu_sc as plsc

B = 16384
EMB = 16
H1, H2, H3 = 16, 8, 4

NC, NS, L = 2, 16, 16
NW = NC * NS
BPW = B // NW
G = BPW // L

OFF_W1 = 0
OFF_B1 = OFF_W1 + 2 * EMB * H1
OFF_W2 = OFF_B1 + H1
OFF_B2 = OFF_W2 + H1 * H2
OFF_W3 = OFF_B2 + H2
OFF_B3 = OFF_W3 + H2 * H3
OFF_WP = OFF_B3 + H3
OFF_BP = OFF_WP + EMB + H3
NPAR = 728


def _tile_body(uid_h, iid_h, uM_h, iM_h, uG_h, iG_h, par_h, out_h,
               uid_v, iid_v, uM_v, iM_v, uG_v, iG_v, par_v, par_s, out_v,
               sem):
    wid = lax.axis_index("s") * NC + lax.axis_index("c")
    base = pl.multiple_of(wid * BPW, BPW)

    pltpu.sync_copy(uid_h.at[pl.ds(base, BPW)], uid_v)
    pltpu.sync_copy(iid_h.at[pl.ds(base, BPW)], iid_v)
    c1 = pltpu.async_copy(uM_h.at[uid_v], uM_v, sem)
    c2 = pltpu.async_copy(iM_h.at[iid_v], iM_v, sem)
    c3 = pltpu.async_copy(uG_h.at[uid_v], uG_v, sem)
    c4 = pltpu.async_copy(iG_h.at[iid_v], iG_v, sem)
    pltpu.sync_copy(par_h, par_v)
    pltpu.sync_copy(par_v, par_s)
    c1.wait()
    c2.wait()
    c3.wait()
    c4.wait()

    iota = lax.iota(jnp.int32, L)

    @plsc.parallel_loop(0, G)
    def group(g):
        start = pl.multiple_of(g * L, L)
        row = start + iota

        cols = [jnp.full((L,), k, jnp.int32) for k in range(EMB)]
        acc1 = [None] * H1
        for k in range(2 * EMB):
            if k < EMB:
                x = plsc.load_gather(uM_v, [row, cols[k]])
            else:
                x = plsc.load_gather(iM_v, [row, cols[k - EMB]])
            for j in range(H1):
                t = x * par_s[OFF_W1 + k * H1 + j]
                acc1[j] = t if acc1[j] is None else acc1[j] + t
        h1 = [jnp.maximum(acc1[j] + par_s[OFF_B1 + j], 0.0)
              for j in range(H1)]

        acc2 = [None] * H2
        for k in range(H1):
            for j in range(H2):
                t = h1[k] * par_s[OFF_W2 + k * H2 + j]
                acc2[j] = t if acc2[j] is None else acc2[j] + t
        h2 = [jnp.maximum(acc2[j] + par_s[OFF_B2 + j], 0.0)
              for j in range(H2)]

        acc3 = [None] * H3
        for k in range(H2):
            for j in range(H3):
                t = h2[k] * par_s[OFF_W3 + k * H3 + j]
                acc3[j] = t if acc3[j] is None else acc3[j] + t
        h3 = [jnp.maximum(acc3[j] + par_s[OFF_B3 + j], 0.0)
              for j in range(H3)]

        z = None
        for k in range(EMB):
            gm = (plsc.load_gather(uG_v, [row, cols[k]])
                  * plsc.load_gather(iG_v, [row, cols[k]]))
            t = gm * par_s[OFF_WP + k]
            z = t if z is None else z + t
        for k in range(H3):
            z = z + h3[k] * par_s[OFF_WP + EMB + k]
        z = z + par_s[OFF_BP]
        out_v[pl.ds(start, L)] = 1.0 / (1.0 + jnp.exp(-z))

    pltpu.sync_copy(out_v, out_h.at[pl.ds(base, BPW)])


@functools.partial(
    pl.kernel,
    mesh=plsc.VectorSubcoreMesh(core_axis_name="c", subcore_axis_name="s"),
    compiler_params=pltpu.CompilerParams(
        needs_layout_passes=False, use_tc_tiling_on_sc=False),
    out_type=jax.ShapeDtypeStruct((B,), jnp.float32),
    scratch_types=[
        pltpu.VMEM((BPW,), jnp.int32),
        pltpu.VMEM((BPW,), jnp.int32),
        pltpu.VMEM((BPW, EMB), jnp.float32),
        pltpu.VMEM((BPW, EMB), jnp.float32),
        pltpu.VMEM((BPW, EMB), jnp.float32),
        pltpu.VMEM((BPW, EMB), jnp.float32),
        pltpu.VMEM_SHARED((NPAR,), jnp.float32),
        pltpu.SMEM((NPAR,), jnp.float32),
        pltpu.VMEM((BPW,), jnp.float32),
        pltpu.SemaphoreType.DMA,
    ],
)
def _neumf_sc(uid_h, iid_h, uM_h, iM_h, uG_h, iG_h, par_h, out_h,
              uid_v, iid_v, uM_v, iM_v, uG_v, iG_v, par_v, par_s, out_v,
              sem):
    _tile_body(uid_h, iid_h, uM_h, iM_h, uG_h, iG_h, par_h, out_h,
               uid_v, iid_v, uM_v, iM_v, uG_v, iG_v, par_v, par_s, out_v,
               sem)


def kernel(userID, itemID, user_emb_MLP, item_emb_MLP, user_emb_GMF,
           item_emb_GMF, W1, b1, W2, b2, W3, b3, Wp, bp):
    uid = userID.reshape(-1).astype(jnp.int32)
    iid = itemID.reshape(-1).astype(jnp.int32)
    params = jnp.concatenate([
        W1.reshape(-1), b1, W2.reshape(-1), b2, W3.reshape(-1), b3,
        Wp.reshape(-1), bp,
        jnp.zeros((NPAR - OFF_BP - 1,), jnp.float32),
    ]).astype(jnp.float32)
    out = _neumf_sc(uid, iid, user_emb_MLP, item_emb_MLP, user_emb_GMF,
                    item_emb_GMF, params)
    return out.reshape(B, 1)
